# baseline (speedup 1.0000x reference)
import functools

import jax
import jax.numpy as jnp
from jax.experimental import pallas as pl
from jax.experimental.pallas import tpu as pltpu
from jax.experimental.pallas import tpu_sc as plsc

T = 2048
D = 1024
H = 16
KV = 4
HD = 128
E = 8
TOPK = 2
DFF = 768
EPS = 1e-6
THETA = 10000.0
NH = H + 2 * KV

MM = jnp.bfloat16

BT1 = 256
BQ = 256
BT3 = 256
BTM = 256
NBLK = TOPK * T // BTM + E
CAP = NBLK * BTM


def _qkv_body(x_ref, win_ref, wqkv_ref, wsel_ref, cos_ref, sin_ref,
              q_ref, k_ref, v_ref):
    x = x_ref[...]
    xn = x * jax.lax.rsqrt(jnp.mean(x * x, axis=-1, keepdims=True) + EPS)
    xn = xn * win_ref[...]
    qkv = jnp.dot(xn.astype(MM), wqkv_ref[...],
                  preferred_element_type=jnp.float32)
    u = qkv.reshape(-1, NH, HD)
    ms = jnp.mean(u * u, axis=-1, keepdims=True)
    un = u * jax.lax.rsqrt(ms + EPS) * wsel_ref[...][None]
    hidx = jax.lax.broadcasted_iota(jnp.int32, (1, NH, 1), 1)
    norm_mask = hidx < (H + KV)
    un = jnp.where(norm_mask, un, u)
    cos = cos_ref[...][:, None, :]
    sin = sin_ref[...][:, None, :]
    u1 = un[..., : HD // 2]
    u2 = un[..., HD // 2:]
    rot = jnp.concatenate([-u2, u1], axis=-1)
    outu = jnp.where(norm_mask, cos * un + sin * rot, un)
    q_ref[...] = outu[:, :H, :].reshape(-1, H * HD).astype(MM)
    k_ref[...] = outu[:, H:H + KV, :].reshape(-1, KV * HD).astype(MM)
    v_ref[...] = outu[:, H + KV:, :].reshape(-1, KV * HD).astype(MM)


def _attn_body(q_ref, k_ref, v_ref, o_ref):
    i = pl.program_id(1)
    q = q_ref[...]
    scale = HD ** -0.5

    for idx in range(T // BQ):
        @pl.when(i == idx)
        def _(idx=idx):
            w = (idx + 1) * BQ
            k = k_ref[:w, :]
            s = jax.lax.dot_general(q, k, (((1,), (1,)), ((), ())),
                                    preferred_element_type=jnp.float32)
            s = s * scale
            row = idx * BQ + jax.lax.broadcasted_iota(jnp.int32, (BQ, w), 0)
            col = jax.lax.broadcasted_iota(jnp.int32, (BQ, w), 1)
            s = jnp.where(col <= row, s, -1e9)
            m = jnp.max(s, axis=-1, keepdims=True)
            p = jnp.exp(s - m)
            denom = jnp.sum(p, axis=-1, keepdims=True)
            o = jnp.dot(p.astype(MM), v_ref[:w, :],
                        preferred_element_type=jnp.float32)
            o_ref[...] = (o / denom).astype(MM)


def _oproj_body(o_ref, wo_ref, res_ref, wpost_ref, wgate_ref,
                h_ref, x2_ref, wf_ref):
    o = o_ref[...]
    h = res_ref[...] + jnp.dot(o, wo_ref[...],
                               preferred_element_type=jnp.float32)
    h_ref[...] = h
    x2 = h * jax.lax.rsqrt(jnp.mean(h * h, axis=-1, keepdims=True) + EPS)
    x2 = x2 * wpost_ref[...]
    x2_ref[...] = x2
    logits = jnp.dot(x2, wgate_ref[...], preferred_element_type=jnp.float32)
    li = jax.lax.broadcasted_iota(jnp.int32, logits.shape, 1)
    m1 = jnp.max(logits, axis=-1, keepdims=True)
    i1 = jnp.min(jnp.where(logits == m1, li, E), axis=-1, keepdims=True)
    oh1 = li == i1
    l2 = jnp.where(oh1, -jnp.inf, logits)
    m2 = jnp.max(l2, axis=-1, keepdims=True)
    i2 = jnp.min(jnp.where(l2 == m2, li, E), axis=-1, keepdims=True)
    oh2 = li == i2
    w1v = 1.0 / (1.0 + jnp.exp(m2 - m1))
    wf_ref[...] = jnp.where(oh1, w1v, 0.0) + jnp.where(oh2, 1.0 - w1v, 0.0)


def _ffn_body(be_ref, xs_ref, sw_ref, w1_ref, w3_ref, w2_ref, eo_ref):
    b = pl.program_id(0)

    @pl.when(be_ref[b] < E)
    def _():
        xs = xs_ref[...].astype(MM)
        g = jnp.dot(xs, w1_ref[0], preferred_element_type=jnp.float32)
        u = jnp.dot(xs, w3_ref[0], preferred_element_type=jnp.float32)
        hm = (g * jax.nn.sigmoid(g) * u).astype(MM)
        eo = jnp.dot(hm, w2_ref[0], preferred_element_type=jnp.float32)
        eo_ref[...] = eo * sw_ref[:, :1]


def _sc_gather(table, idx):
    info = plsc.get_sparse_core_info()
    nw = info.num_cores * info.num_subcores
    per_w = CAP // nw
    chunk = 48
    mesh = plsc.VectorSubcoreMesh(core_axis_name="c", subcore_axis_name="s")

    @functools.partial(
        pl.kernel, mesh=mesh,
        out_type=jax.ShapeDtypeStruct((CAP, D), jnp.float32),
        scratch_types=[
            pltpu.VMEM((chunk,), jnp.int32),
            pltpu.VMEM((chunk, D), jnp.float32),
            pltpu.SemaphoreType.DMA,
        ],
    )
    def k(table_hbm, idx_hbm, out_hbm, idx_v, rows_v, sem):
        wid = jax.lax.axis_index("s") * info.num_cores + jax.lax.axis_index("c")
        base = wid * per_w
        for c in range(per_w // chunk):
            off = base + c * chunk
            pltpu.sync_copy(idx_hbm.at[pl.ds(off, chunk)], idx_v)
            pltpu.async_copy(table_hbm.at[idx_v], rows_v, sem).wait()
            pltpu.sync_copy(rows_v, out_hbm.at[pl.ds(off, chunk)])

    return k(table, idx)


def _sc_combine(h, eo, pos):
    info = plsc.get_sparse_core_info()
    nw = info.num_cores * info.num_subcores
    per_w = T // nw
    ch = 16
    mesh = plsc.VectorSubcoreMesh(core_axis_name="c", subcore_axis_name="s")

    @functools.partial(
        pl.kernel, mesh=mesh,
        out_type=jax.ShapeDtypeStruct((T, D), jnp.float32),
        scratch_types=[
            pltpu.VMEM((2 * ch,), jnp.int32),
            pltpu.VMEM((2 * ch, D), jnp.float32),
            pltpu.VMEM((ch, D), jnp.float32),
            pltpu.SemaphoreType.DMA,
        ],
    )
    def k(h_hbm, eo_hbm, pos_hbm, out_hbm, idx_v, rows_v, h_v, sem):
        wid = jax.lax.axis_index("s") * info.num_cores + jax.lax.axis_index("c")
        for c in range(per_w // ch):
            tb = wid * per_w + c * ch
            pltpu.sync_copy(pos_hbm.at[pl.ds(2 * tb, 2 * ch)], idx_v)
            pltpu.async_copy(eo_hbm.at[idx_v], rows_v, sem).wait()
            pltpu.sync_copy(h_hbm.at[pl.ds(tb, ch)], h_v)

            def tok_body(i, _):
                def lane_body(j, _):
                    sl = pl.ds(j * 16, 16)
                    h_v[i, sl] = (h_v[i, sl] + rows_v[2 * i, sl]
                                  + rows_v[2 * i + 1, sl])
                    return 0
                return jax.lax.fori_loop(0, D // 16, lane_body, 0)

            jax.lax.fori_loop(0, ch, tok_body, 0)
            pltpu.sync_copy(h_v, out_hbm.at[pl.ds(tb, ch)])

    return k(h, eo, pos)


def kernel(hidden_states, positions, w_in, w_qkv, q_norm_w, k_norm_w,
           w_o, w_post, w_gate, w1, w3, w2):
    pos = positions.astype(jnp.float32)
    inv_freq = 1.0 / (THETA ** (jnp.arange(0, HD, 2, dtype=jnp.float32) / HD))
    freqs = pos[:, None] * inv_freq[None, :]
    emb = jnp.concatenate([freqs, freqs], axis=-1)
    cos = jnp.cos(emb)
    sin = jnp.sin(emb)
    wsel = jnp.concatenate([
        jnp.tile(q_norm_w[None], (H, 1)),
        jnp.tile(k_norm_w[None], (KV, 1)),
        jnp.ones((KV, HD), jnp.float32),
    ], axis=0)
    wqkv_b = w_qkv.astype(MM)
    wo_b = w_o.astype(MM)
    w1_b = w1.astype(MM)
    w3_b = w3.astype(MM)
    w2_b = w2.astype(MM)
    win2 = w_in[None]
    wpost2 = w_post[None]

    nt1 = T // BT1
    q, k, v = pl.pallas_call(
        _qkv_body,
        grid=(nt1,),
        in_specs=[
            pl.BlockSpec((BT1, D), lambda i: (i, 0)),
            pl.BlockSpec((1, D), lambda i: (0, 0)),
            pl.BlockSpec((D, NH * HD), lambda i: (0, 0)),
            pl.BlockSpec((NH, HD), lambda i: (0, 0)),
            pl.BlockSpec((BT1, HD), lambda i: (i, 0)),
            pl.BlockSpec((BT1, HD), lambda i: (i, 0)),
        ],
        out_specs=[
            pl.BlockSpec((BT1, H * HD), lambda i: (i, 0)),
            pl.BlockSpec((BT1, KV * HD), lambda i: (i, 0)),
            pl.BlockSpec((BT1, KV * HD), lambda i: (i, 0)),
        ],
        out_shape=[
            jax.ShapeDtypeStruct((T, H * HD), MM),
            jax.ShapeDtypeStruct((T, KV * HD), MM),
            jax.ShapeDtypeStruct((T, KV * HD), MM),
        ],
    )(hidden_states, win2, wqkv_b, wsel, cos, sin)

    grp = H // KV
    o = pl.pallas_call(
        _attn_body,
        grid=(H, T // BQ),
        in_specs=[
            pl.BlockSpec((BQ, HD), lambda h, i: (i, h)),
            pl.BlockSpec((T, HD), lambda h, i: (0, h // grp)),
            pl.BlockSpec((T, HD), lambda h, i: (0, h // grp)),
        ],
        out_specs=pl.BlockSpec((BQ, HD), lambda h, i: (i, h)),
        out_shape=jax.ShapeDtypeStruct((T, H * HD), MM),
        compiler_params=pltpu.CompilerParams(
            dimension_semantics=("arbitrary", "arbitrary")),
    )(q, k, v)

    nt3 = T // BT3
    h, x2f, wf = pl.pallas_call(
        _oproj_body,
        grid=(nt3,),
        in_specs=[
            pl.BlockSpec((BT3, H * HD), lambda i: (i, 0)),
            pl.BlockSpec((H * HD, D), lambda i: (0, 0)),
            pl.BlockSpec((BT3, D), lambda i: (i, 0)),
            pl.BlockSpec((1, D), lambda i: (0, 0)),
            pl.BlockSpec((D, E), lambda i: (0, 0)),
        ],
        out_specs=[
            pl.BlockSpec((BT3, D), lambda i: (i, 0)),
            pl.BlockSpec((BT3, D), lambda i: (i, 0)),
            pl.BlockSpec((BT3, E), lambda i: (i, 0)),
        ],
        out_shape=[
            jax.ShapeDtypeStruct((T, D), jnp.float32),
            jax.ShapeDtypeStruct((T, D), jnp.float32),
            jax.ShapeDtypeStruct((T, E), jnp.float32),
        ],
    )(o, wo_b, hidden_states, wpost2, w_gate)

    i32 = jnp.int32
    maskTE = wf > 0.0
    cnt = jnp.sum(maskTE.astype(i32), axis=0)
    mcol = maskTE.T.reshape(-1)
    rank = jnp.cumsum(mcol.astype(i32)) - 1
    nb_e = (cnt + BTM - 1) // BTM
    blk_off = jnp.concatenate([jnp.zeros((1,), i32), jnp.cumsum(nb_e)])
    g_off = blk_off[:-1] * BTM
    start = jnp.concatenate([jnp.zeros((1,), i32),
                             jnp.cumsum(cnt)])[:-1]
    eidx = (jnp.arange(E * T, dtype=i32) // T)
    tokid = jnp.arange(E * T, dtype=i32) % T
    slot = g_off[eidx] + rank - start[eidx]
    slot_safe = jnp.where(mcol, slot, CAP)
    slot_token = jnp.zeros((CAP,), i32).at[slot_safe].set(tokid, mode='drop')
    slot_w = jnp.zeros((CAP,), jnp.float32).at[slot_safe].set(
        wf.T.reshape(-1), mode='drop')
    sw_bcast = jnp.broadcast_to(slot_w[:, None], (CAP, HD))
    block_expert = jnp.searchsorted(
        blk_off[1:], jnp.arange(NBLK, dtype=i32), side='right').astype(i32)
    kpos = (jnp.cumsum(maskTE.astype(i32), axis=1) - 1).T.reshape(-1)
    pair_idx = jnp.where(mcol, tokid * TOPK + kpos, TOPK * T)
    pos_ab = jnp.zeros((TOPK * T,), i32).at[pair_idx].set(slot, mode='drop')

    xs = _sc_gather(x2f, slot_token)

    grid_spec = pltpu.PrefetchScalarGridSpec(
        num_scalar_prefetch=1,
        grid=(NBLK,),
        in_specs=[
            pl.BlockSpec((BTM, D), lambda b, be: (b, 0)),
            pl.BlockSpec((BTM, HD), lambda b, be: (b, 0)),
            pl.BlockSpec((1, D, DFF),
                         lambda b, be: (jnp.minimum(be[b], E - 1), 0, 0)),
            pl.BlockSpec((1, D, DFF),
                         lambda b, be: (jnp.minimum(be[b], E - 1), 0, 0)),
            pl.BlockSpec((1, DFF, D),
                         lambda b, be: (jnp.minimum(be[b], E - 1), 0, 0)),
        ],
        out_specs=pl.BlockSpec((BTM, D), lambda b, be: (b, 0)),
    )
    eo = pl.pallas_call(
        _ffn_body,
        grid_spec=grid_spec,
        out_shape=jax.ShapeDtypeStruct((CAP, D), jnp.float32),
        compiler_params=pltpu.CompilerParams(
            dimension_semantics=("arbitrary",)),
    )(block_expert, xs, sw_bcast, w1_b, w3_b, w2_b)

    return _sc_combine(h, eo, pos_ab)


# speedup vs baseline: 2.0415x; 2.0415x over previous
import jax
import jax.numpy as jnp
from jax.experimental import pallas as pl
from jax.experimental.pallas import tpu as pltpu

T = 2048
D = 1024
H = 16
KV = 4
HD = 128
E = 8
DFF = 768
EPS = 1e-6
THETA = 10000.0
NH = H + 2 * KV

MM = jnp.bfloat16

BT1 = 256
BQ = 512
BK = 256
BT3 = 256
BT4 = 1024


def _qkv_body(x_ref, win_ref, wqkv_ref, wsel_ref, cos_ref, sin_ref,
              q_ref, k_ref, v_ref):
    x = x_ref[...]
    xn = x * jax.lax.rsqrt(jnp.mean(x * x, axis=-1, keepdims=True) + EPS)
    xn = xn * win_ref[...]
    qkv = jnp.dot(xn.astype(MM), wqkv_ref[...],
                  preferred_element_type=jnp.float32)
    u = qkv.reshape(-1, NH, HD)
    ms = jnp.mean(u * u, axis=-1, keepdims=True)
    un = u * jax.lax.rsqrt(ms + EPS) * wsel_ref[...][None]
    hidx = jax.lax.broadcasted_iota(jnp.int32, (1, NH, 1), 1)
    norm_mask = hidx < (H + KV)
    un = jnp.where(norm_mask, un, u)
    cos = cos_ref[...][:, None, :]
    sin = sin_ref[...][:, None, :]
    u1 = un[..., : HD // 2]
    u2 = un[..., HD // 2:]
    rot = jnp.concatenate([-u2, u1], axis=-1)
    outu = jnp.where(norm_mask, cos * un + sin * rot, un)
    q_ref[...] = outu[:, :H, :].reshape(-1, H * HD).astype(MM)
    k_ref[...] = outu[:, H:H + KV, :].reshape(-1, KV * HD).astype(MM)
    v_ref[...] = outu[:, H + KV:, :].reshape(-1, KV * HD).astype(MM)


def _attn_body(q_ref, k_ref, v_ref, o_ref):
    i = pl.program_id(1)
    q = q_ref[...]
    scale = HD ** -0.5

    for idx in range(T // BQ):
        @pl.when(i == idx)
        def _(idx=idx):
            w = (idx + 1) * BQ
            k = k_ref[:w, :]
            s = jax.lax.dot_general(q, k, (((1,), (1,)), ((), ())),
                                    preferred_element_type=jnp.float32)
            s = s * scale
            row = idx * BQ + jax.lax.broadcasted_iota(jnp.int32, (BQ, w), 0)
            col = jax.lax.broadcasted_iota(jnp.int32, (BQ, w), 1)
            s = jnp.where(col <= row, s, -1e9)
            m = jnp.max(s, axis=-1, keepdims=True)
            p = jnp.exp(s - m)
            denom = jnp.sum(p, axis=-1, keepdims=True)
            o = jnp.dot(p.astype(MM), v_ref[:w, :],
                        preferred_element_type=jnp.float32)
            o_ref[...] = (o / denom).astype(MM)


def _moe_body(o_ref, wo_ref, res_ref, wpost_ref, wgate_ref,
              w1_ref, w3_ref, w2_ref, out_ref, x2_s, wf_s):
    e = pl.program_id(1)

    @pl.when(e == 0)
    def _():
        o = o_ref[...]
        h = res_ref[...] + jnp.dot(o, wo_ref[...],
                                   preferred_element_type=jnp.float32)
        out_ref[...] = h
        x2 = h * jax.lax.rsqrt(jnp.mean(h * h, axis=-1, keepdims=True) + EPS)
        x2 = x2 * wpost_ref[...]
        x2_s[...] = x2.astype(MM)
        logits = jnp.dot(x2, wgate_ref[...],
                         preferred_element_type=jnp.float32)
        li = jax.lax.broadcasted_iota(jnp.int32, logits.shape, 1)
        m1 = jnp.max(logits, axis=-1, keepdims=True)
        i1 = jnp.min(jnp.where(logits == m1, li, E), axis=-1, keepdims=True)
        oh1 = li == i1
        l2 = jnp.where(oh1, -jnp.inf, logits)
        m2 = jnp.max(l2, axis=-1, keepdims=True)
        i2 = jnp.min(jnp.where(l2 == m2, li, E), axis=-1, keepdims=True)
        oh2 = li == i2
        w1v = 1.0 / (1.0 + jnp.exp(m2 - m1))
        wf_s[...] = jnp.where(oh1, w1v, 0.0) + jnp.where(oh2, 1.0 - w1v, 0.0)

    @pl.when(e != 0)
    def _():
        x2 = x2_s[...]
        g = jnp.dot(x2, w1_ref[0], preferred_element_type=jnp.float32)
        u = jnp.dot(x2, w3_ref[0], preferred_element_type=jnp.float32)
        hm = (g * jax.nn.sigmoid(g) * u).astype(MM)
        eo = jnp.dot(hm, w2_ref[0], preferred_element_type=jnp.float32)
        li = jax.lax.broadcasted_iota(jnp.int32, wf_s.shape, 1)
        wcol = jnp.sum(jnp.where(li == e - 1, wf_s[...], 0.0), axis=-1,
                       keepdims=True)
        out_ref[...] = out_ref[...] + eo * wcol


def kernel(hidden_states, positions, w_in, w_qkv, q_norm_w, k_norm_w,
           w_o, w_post, w_gate, w1, w3, w2):
    pos = positions.astype(jnp.float32)
    inv_freq = 1.0 / (THETA ** (jnp.arange(0, HD, 2, dtype=jnp.float32) / HD))
    freqs = pos[:, None] * inv_freq[None, :]
    emb = jnp.concatenate([freqs, freqs], axis=-1)
    cos = jnp.cos(emb)
    sin = jnp.sin(emb)
    wsel = jnp.concatenate([
        jnp.tile(q_norm_w[None], (H, 1)),
        jnp.tile(k_norm_w[None], (KV, 1)),
        jnp.ones((KV, HD), jnp.float32),
    ], axis=0)
    wqkv_b = w_qkv.astype(MM)
    wo_b = w_o.astype(MM)
    w1_b = w1.astype(MM)
    w3_b = w3.astype(MM)
    w2_b = w2.astype(MM)
    win2 = w_in[None]
    wpost2 = w_post[None]

    nt1 = T // BT1
    q, k, v = pl.pallas_call(
        _qkv_body,
        grid=(nt1,),
        in_specs=[
            pl.BlockSpec((BT1, D), lambda i: (i, 0)),
            pl.BlockSpec((1, D), lambda i: (0, 0)),
            pl.BlockSpec((D, NH * HD), lambda i: (0, 0)),
            pl.BlockSpec((NH, HD), lambda i: (0, 0)),
            pl.BlockSpec((BT1, HD), lambda i: (i, 0)),
            pl.BlockSpec((BT1, HD), lambda i: (i, 0)),
        ],
        out_specs=[
            pl.BlockSpec((BT1, H * HD), lambda i: (i, 0)),
            pl.BlockSpec((BT1, KV * HD), lambda i: (i, 0)),
            pl.BlockSpec((BT1, KV * HD), lambda i: (i, 0)),
        ],
        out_shape=[
            jax.ShapeDtypeStruct((T, H * HD), MM),
            jax.ShapeDtypeStruct((T, KV * HD), MM),
            jax.ShapeDtypeStruct((T, KV * HD), MM),
        ],
    )(hidden_states, win2, wqkv_b, wsel, cos, sin)

    grp = H // KV
    o = pl.pallas_call(
        _attn_body,
        grid=(H, T // BQ),
        in_specs=[
            pl.BlockSpec((BQ, HD), lambda h, i: (i, h)),
            pl.BlockSpec((T, HD), lambda h, i: (0, h // grp)),
            pl.BlockSpec((T, HD), lambda h, i: (0, h // grp)),
        ],
        out_specs=pl.BlockSpec((BQ, HD), lambda h, i: (i, h)),
        out_shape=jax.ShapeDtypeStruct((T, H * HD), MM),
        compiler_params=pltpu.CompilerParams(
            dimension_semantics=("arbitrary", "arbitrary")),
    )(q, k, v)

    nt4 = T // BT4
    out = pl.pallas_call(
        _moe_body,
        grid=(nt4, E + 1),
        in_specs=[
            pl.BlockSpec((BT4, H * HD), lambda t, e: (t, 0)),
            pl.BlockSpec((H * HD, D), lambda t, e: (0, 0)),
            pl.BlockSpec((BT4, D), lambda t, e: (t, 0)),
            pl.BlockSpec((1, D), lambda t, e: (0, 0)),
            pl.BlockSpec((D, E), lambda t, e: (0, 0)),
            pl.BlockSpec((1, D, DFF), lambda t, e: (jnp.maximum(e - 1, 0), 0, 0)),
            pl.BlockSpec((1, D, DFF), lambda t, e: (jnp.maximum(e - 1, 0), 0, 0)),
            pl.BlockSpec((1, DFF, D), lambda t, e: (jnp.maximum(e - 1, 0), 0, 0)),
        ],
        out_specs=pl.BlockSpec((BT4, D), lambda t, e: (t, 0)),
        out_shape=jax.ShapeDtypeStruct((T, D), jnp.float32),
        scratch_shapes=[
            pltpu.VMEM((BT4, D), MM),
            pltpu.VMEM((BT4, E), jnp.float32),
        ],
        compiler_params=pltpu.CompilerParams(
            dimension_semantics=("parallel", "arbitrary")),
    )(o, wo_b, hidden_states, wpost2, w_gate, w1_b, w3_b, w2_b)

    return out
